# baseline (speedup 1.0000x reference)
import functools

import jax
import jax.numpy as jnp
from jax import lax
from jax.experimental import pallas as pl
from jax.experimental.pallas import tpu as pltpu
from jax.experimental.pallas import tpu_sc as plsc

N = 10000
M = 10000
E = 320000
F = 128
NEG = 0.2
NC, NS = 2, 16
NW = NC * NS
NR = 80
EPT = NR * 128
E_PAD = NW * EPT
ER = E_PAD // 128
NPAD = 10240
CHUNK = NPAD // NS
STP = 20480
F32 = jnp.float32
I32 = jnp.int32


def _mesh():
    return plsc.VectorSubcoreMesh(core_axis_name="c", subcore_axis_name="s")


def _wid_r0_off():
    cid = lax.axis_index("c")
    sid = lax.axis_index("s")
    wid = cid * NS + sid
    return cid, sid, wid, wid * NR, sid * CHUNK


def _tc_proj_body(x_ref, w_ref, att_ref, xp_ref, st_ref):
    xp = jnp.dot(x_ref[...], w_ref[...], preferred_element_type=F32)
    xp_ref[...] = xp
    st_ref[...] = lax.dot_general(
        xp, att_ref[...], (((1,), (1,)), ((), ())), preferred_element_type=F32
    )


def _tc_proj(x2, w, att2):
    return pl.pallas_call(
        _tc_proj_body,
        out_shape=(
            jax.ShapeDtypeStruct((N, F), F32),
            jax.ShapeDtypeStruct((N, 2), F32),
        ),
    )(x2, w, att2)


def _k1_body(nidx_h, eidx_h, st_h, ones_h, zeros_h,
             sep_h, dp_h, bp_h,
             idxn_v, idxe_v, st_v, ones_v, tv_v, se_sh, d_sh, b_sh):
    cid, sid, wid, r0, off = _wid_r0_off()
    zsl = pl.ds(off, CHUNK)
    pltpu.sync_copy(zeros_h.at[zsl], se_sh.at[zsl])
    pltpu.sync_copy(zeros_h.at[zsl], d_sh.at[zsl])
    pltpu.sync_copy(zeros_h.at[zsl], b_sh.at[zsl])
    pltpu.sync_copy(nidx_h.at[pl.ds(r0, NR), :], idxn_v)
    pltpu.sync_copy(eidx_h.at[pl.ds(r0, NR), :], idxe_v)
    pltpu.sync_copy(st_h, st_v)
    pltpu.sync_copy(ones_h.at[pl.ds(r0, NR), :], ones_v)

    def body(i, carry):
        for q in range(8):
            sl = pl.ds(q * 16, 16)
            iv = idxn_v[i, sl]
            tv = plsc.load_gather(st_v, [iv * 2 + 1])
            tv_v[i, sl] = tv * ones_v[i, sl]
        return carry

    lax.fori_loop(0, NR, body, 0)
    plsc.subcore_barrier()

    def scat(i, carry):
        pltpu.sync_copy(tv_v.at[i], se_sh.at[idxe_v.at[i]], add=True)
        pltpu.sync_copy(ones_v.at[i], d_sh.at[idxn_v.at[i]], add=True)
        pltpu.sync_copy(ones_v.at[i], b_sh.at[idxe_v.at[i]], add=True)
        return carry

    lax.fori_loop(0, NR, scat, 0)
    plsc.subcore_barrier()
    osl = pl.ds(cid * NPAD + off, CHUNK)
    pltpu.sync_copy(se_sh.at[zsl], sep_h.at[osl])
    pltpu.sync_copy(d_sh.at[zsl], dp_h.at[osl])
    pltpu.sync_copy(b_sh.at[zsl], bp_h.at[osl])


def _k1(nidx2, eidx2, st_flat, ones2, zeros1):
    return pl.kernel(
        _k1_body,
        out_type=(
            jax.ShapeDtypeStruct((NC * NPAD,), F32),
            jax.ShapeDtypeStruct((NC * NPAD,), F32),
            jax.ShapeDtypeStruct((NC * NPAD,), F32),
        ),
        mesh=_mesh(),
        compiler_params=pltpu.CompilerParams(needs_layout_passes=False),
        scratch_types=[
            pltpu.VMEM((NR, 128), I32),
            pltpu.VMEM((NR, 128), I32),
            pltpu.VMEM((STP,), F32),
            pltpu.VMEM((NR, 128), F32),
            pltpu.VMEM((NR, 128), F32),
            pltpu.VMEM_SHARED((NPAD,), F32),
            pltpu.VMEM_SHARED((NPAD,), F32),
            pltpu.VMEM_SHARED((NPAD,), F32),
        ],
    )(nidx2, eidx2, st_flat, ones2, zeros1)


def _k2_body(nidx_h, eidx_h, st_h, ones_h, zeros_h, sep_h,
             eexp_h, denp_h,
             idxn_v, idxe_v, st_v, ones_v, se0_v, se1_v, ex_v, den_sh):
    cid, sid, wid, r0, off = _wid_r0_off()
    zsl = pl.ds(off, CHUNK)
    pltpu.sync_copy(zeros_h.at[zsl], den_sh.at[zsl])
    pltpu.sync_copy(nidx_h.at[pl.ds(r0, NR), :], idxn_v)
    pltpu.sync_copy(eidx_h.at[pl.ds(r0, NR), :], idxe_v)
    pltpu.sync_copy(st_h, st_v)
    pltpu.sync_copy(ones_h.at[pl.ds(r0, NR), :], ones_v)
    pltpu.sync_copy(sep_h.at[pl.ds(0, NPAD)], se0_v)
    pltpu.sync_copy(sep_h.at[pl.ds(NPAD, NPAD)], se1_v)

    def comb(i, carry):
        sl = pl.ds(i * 16, 16)
        se0_v[sl] = se0_v[sl] + se1_v[sl]
        return carry

    lax.fori_loop(0, NPAD // 16, comb, 0)

    def body(i, carry):
        for q in range(8):
            sl = pl.ds(q * 16, 16)
            ivn = idxn_v[i, sl]
            ive = idxe_v[i, sl]
            s = plsc.load_gather(st_v, [ivn * 2])
            se = plsc.load_gather(se0_v, [ive])
            e = s + se
            e = jnp.where(e > 0, e, NEG * e)
            ex_v[i, sl] = jnp.exp(e) * ones_v[i, sl]
        return carry

    lax.fori_loop(0, NR, body, 0)
    plsc.subcore_barrier()

    def scat(i, carry):
        pltpu.sync_copy(ex_v.at[i], den_sh.at[idxn_v.at[i]], add=True)
        return carry

    lax.fori_loop(0, NR, scat, 0)
    plsc.subcore_barrier()
    pltpu.sync_copy(ex_v, eexp_h.at[pl.ds(r0, NR), :])
    pltpu.sync_copy(den_sh.at[zsl], denp_h.at[pl.ds(cid * NPAD + off, CHUNK)])


def _k2(nidx2, eidx2, st_flat, ones2, zeros1, sep):
    return pl.kernel(
        _k2_body,
        out_type=(
            jax.ShapeDtypeStruct((ER, 128), F32),
            jax.ShapeDtypeStruct((NC * NPAD,), F32),
        ),
        mesh=_mesh(),
        compiler_params=pltpu.CompilerParams(needs_layout_passes=False),
        scratch_types=[
            pltpu.VMEM((NR, 128), I32),
            pltpu.VMEM((NR, 128), I32),
            pltpu.VMEM((STP,), F32),
            pltpu.VMEM((NR, 128), F32),
            pltpu.VMEM((NPAD,), F32),
            pltpu.VMEM((NPAD,), F32),
            pltpu.VMEM((NR, 128), F32),
            pltpu.VMEM_SHARED((NPAD,), F32),
        ],
    )(nidx2, eidx2, st_flat, ones2, zeros1, sep)


def _k2b_body(nidx_h, eexp_h, denp_h,
              alpha_h,
              idxn_v, den0_v, den1_v, al_v):
    cid, sid, wid, r0, off = _wid_r0_off()
    pltpu.sync_copy(nidx_h.at[pl.ds(r0, NR), :], idxn_v)
    pltpu.sync_copy(denp_h.at[pl.ds(0, NPAD)], den0_v)
    pltpu.sync_copy(denp_h.at[pl.ds(NPAD, NPAD)], den1_v)

    def comb(i, carry):
        sl = pl.ds(i * 16, 16)
        d = den0_v[sl] + den1_v[sl]
        den0_v[sl] = 1.0 / jnp.maximum(d, 1e-16)
        return carry

    lax.fori_loop(0, NPAD // 16, comb, 0)
    pltpu.sync_copy(eexp_h.at[pl.ds(r0, NR), :], al_v)

    def abody(i, carry):
        for q in range(8):
            sl = pl.ds(q * 16, 16)
            ivn = idxn_v[i, sl]
            al_v[i, sl] = al_v[i, sl] * plsc.load_gather(den0_v, [ivn])
        return carry

    lax.fori_loop(0, NR, abody, 0)
    pltpu.sync_copy(al_v, alpha_h.at[pl.ds(r0, NR), :])


def _k2b(nidx2, eexp2, denp):
    return pl.kernel(
        _k2b_body,
        out_type=jax.ShapeDtypeStruct((ER, 128), F32),
        mesh=_mesh(),
        compiler_params=pltpu.CompilerParams(needs_layout_passes=False),
        scratch_types=[
            pltpu.VMEM((NR, 128), I32),
            pltpu.VMEM((NPAD,), F32),
            pltpu.VMEM((NPAD,), F32),
            pltpu.VMEM((NR, 128), F32),
        ],
    )(nidx2, eexp2, denp)


CH = 64
NB = 4
SGC = 16
NCH = EPT // CH
NSC = NCH // SGC
ER64 = E_PAD // CH


def _row_pass_body(gidx_h, sidx_h, alpha_h, table_h, zbig_h,
                   out_h,
                   gi_v, si_v, al_v,
                   rows0_v, rows1_v, rows2_v, rows3_v, acc_sh,
                   gs0, gs1, gs2, gs3, ss0, ss1, ss2, ss3):
    cid, sid, wid, r0, off = _wid_r0_off()
    del r0
    c0 = wid * NCH
    zsl = pl.ds(off, CHUNK)
    pltpu.sync_copy(zbig_h.at[zsl, :], acc_sh.at[zsl, :])
    plsc.subcore_barrier()
    rows = (rows0_v, rows1_v, rows2_v, rows3_v)
    gsems = (gs0, gs1, gs2, gs3)
    ssems = (ss0, ss1, ss2, ss3)

    def wait_scatter(b):
        pltpu.make_async_copy(rows[b], acc_sh.at[si_v.at[0]], ssems[b]).wait()

    def issue_gather(j):
        pltpu.async_copy(table_h.at[gi_v.at[j]], rows[j % NB], gsems[j % NB])

    def sbody(g, carry):
        @pl.when(g > 0)
        def _():
            for b in range(NB):
                wait_scatter(b)

        rg = c0 + g * SGC
        pltpu.sync_copy(gidx_h.at[pl.ds(rg, SGC), :], gi_v)
        pltpu.sync_copy(sidx_h.at[pl.ds(rg, SGC), :], si_v)
        pltpu.sync_copy(alpha_h.at[pl.ds(rg, SGC), :], al_v)
        for j in range(NB - 1):
            issue_gather(j)
        for j in range(SGC):
            b = j % NB
            if j + NB - 1 < SGC:
                if j >= 1:
                    wait_scatter((j - 1) % NB)
                issue_gather(j + NB - 1)
            pltpu.make_async_copy(
                table_h.at[gi_v.at[j]], rows[b], gsems[b]).wait()

            def scale(r, c2):
                w = plsc.load_gather(
                    al_v, [jnp.full((16,), j, I32), jnp.full((16,), r, I32)])
                for q in range(8):
                    sl = pl.ds(q * 16, 16)
                    rows[b][r, sl] = rows[b][r, sl] * w
                return c2

            lax.fori_loop(0, CH, scale, 0)
            pltpu.async_copy(rows[b], acc_sh.at[si_v.at[j]], ssems[b],
                             add=True)
        return carry

    lax.fori_loop(0, NSC, sbody, 0)
    for b in range(NB):
        wait_scatter(b)
    plsc.subcore_barrier()
    pltpu.sync_copy(acc_sh.at[zsl, :], out_h.at[cid, zsl, :])


def _k34(gidx64, sidx64, alpha64, table, zbig):
    return pl.kernel(
        _row_pass_body,
        out_type=jax.ShapeDtypeStruct((NC, NPAD, F), F32),
        mesh=_mesh(),
        compiler_params=pltpu.CompilerParams(needs_layout_passes=False),
        scratch_types=[
            pltpu.VMEM((SGC, CH), I32),
            pltpu.VMEM((SGC, CH), I32),
            pltpu.VMEM((SGC, CH), F32),
            pltpu.VMEM((CH, F), F32),
            pltpu.VMEM((CH, F), F32),
            pltpu.VMEM((CH, F), F32),
            pltpu.VMEM((CH, F), F32),
            pltpu.VMEM_SHARED((NPAD, F), F32),
            pltpu.SemaphoreType.DMA,
            pltpu.SemaphoreType.DMA,
            pltpu.SemaphoreType.DMA,
            pltpu.SemaphoreType.DMA,
            pltpu.SemaphoreType.DMA,
            pltpu.SemaphoreType.DMA,
            pltpu.SemaphoreType.DMA,
            pltpu.SemaphoreType.DMA,
        ],
    )(gidx64, sidx64, alpha64, table, zbig)


def _tc_comb_body(part_ref, deg_ref, out_ref):
    deg = deg_ref[pl.ds(0, NPAD)] + deg_ref[pl.ds(NPAD, NPAD)]
    inv = jnp.where(deg > 0, 1.0 / deg, 0.0)
    out_ref[...] = (part_ref[0] + part_ref[1]) * inv[:, None]


def _tc_comb(part, degp):
    return pl.pallas_call(
        _tc_comb_body,
        out_shape=jax.ShapeDtypeStruct((NPAD, F), F32),
    )(part, degp)


def kernel(x, hyperedge_index, W, att):
    x2 = x.reshape(N, F)
    att2 = att.reshape(2, F)
    pad = E_PAD - E
    nidx2 = jnp.concatenate(
        [hyperedge_index[0], jnp.zeros((pad,), I32)]).reshape(ER, 128)
    eidx2 = jnp.concatenate(
        [hyperedge_index[1], jnp.zeros((pad,), I32)]).reshape(ER, 128)
    ones2 = jnp.concatenate(
        [jnp.ones((E,), F32), jnp.zeros((pad,), F32)]).reshape(ER, 128)
    zeros1 = jnp.zeros((NPAD,), F32)
    zbig = jnp.zeros((NPAD, F), F32)

    xp, st = _tc_proj(x2, W, att2)
    st_flat = jnp.concatenate(
        [st.reshape(2 * N), jnp.zeros((STP - 2 * N,), F32)])
    sep, dp, bp = _k1(nidx2, eidx2, st_flat, ones2, zeros1)
    eexp2, denp = _k2(nidx2, eidx2, st_flat, ones2, zeros1, sep)
    alpha2 = _k2b(nidx2, eexp2, denp)
    nidx64 = nidx2.reshape(ER64, CH)
    eidx64 = eidx2.reshape(ER64, CH)
    alpha64 = alpha2.reshape(ER64, CH)
    xep = _k34(nidx64, eidx64, alpha64, xp, zbig)
    xedge = _tc_comb(xep, bp)
    xnp_ = _k34(eidx64, nidx64, alpha64, xedge, zbig)
    xnode = _tc_comb(xnp_, dp)
    return xnode[:N].reshape(1, N, F)


# speedup vs baseline: 1.0011x; 1.0011x over previous
import functools

import jax
import jax.numpy as jnp
from jax import lax
from jax.experimental import pallas as pl
from jax.experimental.pallas import tpu as pltpu
from jax.experimental.pallas import tpu_sc as plsc

N = 10000
M = 10000
E = 320000
F = 128
NEG = 0.2
NC, NS = 2, 16
NW = NC * NS
NR = 80
EPT = NR * 128
E_PAD = NW * EPT
ER = E_PAD // 128
NPAD = 10240
CHUNK = NPAD // NS
STP = 20480
F32 = jnp.float32
I32 = jnp.int32


def _mesh():
    return plsc.VectorSubcoreMesh(core_axis_name="c", subcore_axis_name="s")


def _wid_r0_off():
    cid = lax.axis_index("c")
    sid = lax.axis_index("s")
    wid = cid * NS + sid
    return cid, sid, wid, wid * NR, sid * CHUNK


def _tc_proj_body(x_ref, w_ref, att_ref, xp_ref, st_ref):
    xp = jnp.dot(x_ref[...], w_ref[...], preferred_element_type=F32)
    xp_ref[...] = xp
    st_ref[...] = lax.dot_general(
        xp, att_ref[...], (((1,), (1,)), ((), ())), preferred_element_type=F32
    )


def _tc_proj(x2, w, att2):
    return pl.pallas_call(
        _tc_proj_body,
        out_shape=(
            jax.ShapeDtypeStruct((N, F), F32),
            jax.ShapeDtypeStruct((N, 2), F32),
        ),
    )(x2, w, att2)


def _k1_body(nidx_h, eidx_h, st_h, ones_h, zeros_h,
             sep_h, dp_h, bp_h,
             idxn_v, idxe_v, st_v, ones_v, tv_v, se_sh, d_sh, b_sh):
    cid, sid, wid, r0, off = _wid_r0_off()
    zsl = pl.ds(off, CHUNK)
    pltpu.sync_copy(zeros_h.at[zsl], se_sh.at[zsl])
    pltpu.sync_copy(zeros_h.at[zsl], d_sh.at[zsl])
    pltpu.sync_copy(zeros_h.at[zsl], b_sh.at[zsl])
    pltpu.sync_copy(nidx_h.at[pl.ds(r0, NR), :], idxn_v)
    pltpu.sync_copy(eidx_h.at[pl.ds(r0, NR), :], idxe_v)
    pltpu.sync_copy(st_h, st_v)
    pltpu.sync_copy(ones_h.at[pl.ds(r0, NR), :], ones_v)

    def body(i, carry):
        for q in range(8):
            sl = pl.ds(q * 16, 16)
            iv = idxn_v[i, sl]
            tv = plsc.load_gather(st_v, [iv * 2 + 1])
            tv_v[i, sl] = tv * ones_v[i, sl]
        return carry

    lax.fori_loop(0, NR, body, 0)
    plsc.subcore_barrier()

    def scat(i, carry):
        pltpu.sync_copy(tv_v.at[i], se_sh.at[idxe_v.at[i]], add=True)
        pltpu.sync_copy(ones_v.at[i], d_sh.at[idxn_v.at[i]], add=True)
        pltpu.sync_copy(ones_v.at[i], b_sh.at[idxe_v.at[i]], add=True)
        return carry

    lax.fori_loop(0, NR, scat, 0)
    plsc.subcore_barrier()
    osl = pl.ds(cid * NPAD + off, CHUNK)
    pltpu.sync_copy(se_sh.at[zsl], sep_h.at[osl])
    pltpu.sync_copy(d_sh.at[zsl], dp_h.at[osl])
    pltpu.sync_copy(b_sh.at[zsl], bp_h.at[osl])


def _k1(nidx2, eidx2, st_flat, ones2, zeros1):
    return pl.kernel(
        _k1_body,
        out_type=(
            jax.ShapeDtypeStruct((NC * NPAD,), F32),
            jax.ShapeDtypeStruct((NC * NPAD,), F32),
            jax.ShapeDtypeStruct((NC * NPAD,), F32),
        ),
        mesh=_mesh(),
        compiler_params=pltpu.CompilerParams(needs_layout_passes=False),
        scratch_types=[
            pltpu.VMEM((NR, 128), I32),
            pltpu.VMEM((NR, 128), I32),
            pltpu.VMEM((STP,), F32),
            pltpu.VMEM((NR, 128), F32),
            pltpu.VMEM((NR, 128), F32),
            pltpu.VMEM_SHARED((NPAD,), F32),
            pltpu.VMEM_SHARED((NPAD,), F32),
            pltpu.VMEM_SHARED((NPAD,), F32),
        ],
    )(nidx2, eidx2, st_flat, ones2, zeros1)


def _k2_body(nidx_h, eidx_h, st_h, ones_h, zeros_h, sep_h,
             eexp_h, denp_h,
             idxn_v, idxe_v, st_v, ones_v, se0_v, se1_v, ex_v, den_sh):
    cid, sid, wid, r0, off = _wid_r0_off()
    zsl = pl.ds(off, CHUNK)
    pltpu.sync_copy(zeros_h.at[zsl], den_sh.at[zsl])
    pltpu.sync_copy(nidx_h.at[pl.ds(r0, NR), :], idxn_v)
    pltpu.sync_copy(eidx_h.at[pl.ds(r0, NR), :], idxe_v)
    pltpu.sync_copy(st_h, st_v)
    pltpu.sync_copy(ones_h.at[pl.ds(r0, NR), :], ones_v)
    pltpu.sync_copy(sep_h.at[pl.ds(0, NPAD)], se0_v)
    pltpu.sync_copy(sep_h.at[pl.ds(NPAD, NPAD)], se1_v)

    def comb(i, carry):
        sl = pl.ds(i * 16, 16)
        se0_v[sl] = se0_v[sl] + se1_v[sl]
        return carry

    lax.fori_loop(0, NPAD // 16, comb, 0)

    def body(i, carry):
        for q in range(8):
            sl = pl.ds(q * 16, 16)
            ivn = idxn_v[i, sl]
            ive = idxe_v[i, sl]
            s = plsc.load_gather(st_v, [ivn * 2])
            se = plsc.load_gather(se0_v, [ive])
            e = s + se
            e = jnp.where(e > 0, e, NEG * e)
            ex_v[i, sl] = jnp.exp(e) * ones_v[i, sl]
        return carry

    lax.fori_loop(0, NR, body, 0)
    plsc.subcore_barrier()

    def scat(i, carry):
        pltpu.sync_copy(ex_v.at[i], den_sh.at[idxn_v.at[i]], add=True)
        return carry

    lax.fori_loop(0, NR, scat, 0)
    plsc.subcore_barrier()
    pltpu.sync_copy(ex_v, eexp_h.at[pl.ds(r0, NR), :])
    pltpu.sync_copy(den_sh.at[zsl], denp_h.at[pl.ds(cid * NPAD + off, CHUNK)])


def _k2(nidx2, eidx2, st_flat, ones2, zeros1, sep):
    return pl.kernel(
        _k2_body,
        out_type=(
            jax.ShapeDtypeStruct((ER, 128), F32),
            jax.ShapeDtypeStruct((NC * NPAD,), F32),
        ),
        mesh=_mesh(),
        compiler_params=pltpu.CompilerParams(needs_layout_passes=False),
        scratch_types=[
            pltpu.VMEM((NR, 128), I32),
            pltpu.VMEM((NR, 128), I32),
            pltpu.VMEM((STP,), F32),
            pltpu.VMEM((NR, 128), F32),
            pltpu.VMEM((NPAD,), F32),
            pltpu.VMEM((NPAD,), F32),
            pltpu.VMEM((NR, 128), F32),
            pltpu.VMEM_SHARED((NPAD,), F32),
        ],
    )(nidx2, eidx2, st_flat, ones2, zeros1, sep)


def _k2b_body(nidx_h, eexp_h, denp_h,
              alpha_h,
              idxn_v, den0_v, den1_v, al_v):
    cid, sid, wid, r0, off = _wid_r0_off()
    pltpu.sync_copy(nidx_h.at[pl.ds(r0, NR), :], idxn_v)
    pltpu.sync_copy(denp_h.at[pl.ds(0, NPAD)], den0_v)
    pltpu.sync_copy(denp_h.at[pl.ds(NPAD, NPAD)], den1_v)

    def comb(i, carry):
        sl = pl.ds(i * 16, 16)
        d = den0_v[sl] + den1_v[sl]
        den0_v[sl] = 1.0 / jnp.maximum(d, 1e-16)
        return carry

    lax.fori_loop(0, NPAD // 16, comb, 0)
    pltpu.sync_copy(eexp_h.at[pl.ds(r0, NR), :], al_v)

    def abody(i, carry):
        for q in range(8):
            sl = pl.ds(q * 16, 16)
            ivn = idxn_v[i, sl]
            al_v[i, sl] = al_v[i, sl] * plsc.load_gather(den0_v, [ivn])
        return carry

    lax.fori_loop(0, NR, abody, 0)
    pltpu.sync_copy(al_v, alpha_h.at[pl.ds(r0, NR), :])


def _k2b(nidx2, eexp2, denp):
    return pl.kernel(
        _k2b_body,
        out_type=jax.ShapeDtypeStruct((ER, 128), F32),
        mesh=_mesh(),
        compiler_params=pltpu.CompilerParams(needs_layout_passes=False),
        scratch_types=[
            pltpu.VMEM((NR, 128), I32),
            pltpu.VMEM((NPAD,), F32),
            pltpu.VMEM((NPAD,), F32),
            pltpu.VMEM((NR, 128), F32),
        ],
    )(nidx2, eexp2, denp)


CH = 64
NB = 4
SGC = 16
NCH = EPT // CH
NSC = NCH // SGC
ER64 = E_PAD // CH
FH = F // 2


def _row_pass_body(gidx_h, sidx_h, alpha_h, table_h, zbig_h,
                   out_h,
                   gi_v, si_v, al_v,
                   rows0_v, rows1_v, rows2_v, rows3_v, acc_sh,
                   gs0, gs1, gs2, gs3, ss0, ss1, ss2, ss3):
    cid, sid, wid, r0, off = _wid_r0_off()
    del r0
    c0 = wid * NCH
    zsl = pl.ds(off, CHUNK)
    pltpu.sync_copy(zbig_h.at[zsl, :], acc_sh.at[zsl, :])
    plsc.subcore_barrier()
    rows = (rows0_v, rows1_v, rows2_v, rows3_v)
    gsems = (gs0, gs1, gs2, gs3)
    ssems = (ss0, ss1, ss2, ss3)

    def wait_scatter(b):
        pltpu.make_async_copy(rows[b], acc_sh.at[si_v.at[0]], ssems[b]).wait()

    def issue_gather(j):
        pltpu.async_copy(table_h.at[gi_v.at[j]], rows[j % NB], gsems[j % NB])

    def sbody(g, carry):
        @pl.when(g > 0)
        def _():
            for b in range(NB):
                wait_scatter(b)

        rg = c0 + g * SGC
        pltpu.sync_copy(gidx_h.at[pl.ds(rg, SGC), :], gi_v)
        pltpu.sync_copy(sidx_h.at[pl.ds(rg, SGC), :], si_v)
        pltpu.sync_copy(alpha_h.at[pl.ds(rg, SGC), :], al_v)
        for j in range(NB - 1):
            issue_gather(j)
        for j in range(SGC):
            b = j % NB
            if j + NB - 1 < SGC:
                if j >= 1:
                    wait_scatter((j - 1) % NB)
                issue_gather(j + NB - 1)
            pltpu.make_async_copy(
                table_h.at[gi_v.at[j]], rows[b], gsems[b]).wait()

            def scale(r, c2):
                w = plsc.load_gather(
                    al_v, [jnp.full((16,), j, I32), jnp.full((16,), r, I32)])
                for q in range(FH // 16):
                    sl = pl.ds(q * 16, 16)
                    rows[b][r, sl] = rows[b][r, sl] * w
                return c2

            lax.fori_loop(0, CH, scale, 0)
            pltpu.async_copy(rows[b], acc_sh.at[si_v.at[j]], ssems[b],
                             add=True)
        return carry

    lax.fori_loop(0, NSC, sbody, 0)
    for b in range(NB):
        wait_scatter(b)
    plsc.subcore_barrier()
    pltpu.sync_copy(acc_sh.at[zsl, :], out_h.at[cid, zsl, :])


def _k34(gidx64, sidx64, alpha64, table, zbig):
    return pl.kernel(
        _row_pass_body,
        out_type=jax.ShapeDtypeStruct((NC, NPAD, FH), F32),
        mesh=_mesh(),
        compiler_params=pltpu.CompilerParams(
            needs_layout_passes=False, use_tc_tiling_on_sc=False),
        scratch_types=[
            pltpu.VMEM((SGC, CH), I32),
            pltpu.VMEM((SGC, CH), I32),
            pltpu.VMEM((SGC, CH), F32),
            pltpu.VMEM((CH, FH), F32),
            pltpu.VMEM((CH, FH), F32),
            pltpu.VMEM((CH, FH), F32),
            pltpu.VMEM((CH, FH), F32),
            pltpu.VMEM_SHARED((NPAD, FH), F32),
            pltpu.SemaphoreType.DMA,
            pltpu.SemaphoreType.DMA,
            pltpu.SemaphoreType.DMA,
            pltpu.SemaphoreType.DMA,
            pltpu.SemaphoreType.DMA,
            pltpu.SemaphoreType.DMA,
            pltpu.SemaphoreType.DMA,
            pltpu.SemaphoreType.DMA,
        ],
    )(gidx64, sidx64, alpha64, table, zbig)


def _tc_comb_body(part_ref, deg_ref, out_ref):
    deg = deg_ref[pl.ds(0, NPAD)] + deg_ref[pl.ds(NPAD, NPAD)]
    inv = jnp.where(deg > 0, 1.0 / deg, 0.0)
    out_ref[...] = (part_ref[0] + part_ref[1]) * inv[:, None]


def _tc_comb(part, degp):
    return pl.pallas_call(
        _tc_comb_body,
        out_shape=jax.ShapeDtypeStruct((NPAD, part.shape[-1]), F32),
    )(part, degp)


def kernel(x, hyperedge_index, W, att):
    x2 = x.reshape(N, F)
    att2 = att.reshape(2, F)
    pad = E_PAD - E
    nidx2 = jnp.concatenate(
        [hyperedge_index[0], jnp.zeros((pad,), I32)]).reshape(ER, 128)
    eidx2 = jnp.concatenate(
        [hyperedge_index[1], jnp.zeros((pad,), I32)]).reshape(ER, 128)
    ones2 = jnp.concatenate(
        [jnp.ones((E,), F32), jnp.zeros((pad,), F32)]).reshape(ER, 128)
    zeros1 = jnp.zeros((NPAD,), F32)
    zbig = jnp.zeros((NPAD, F), F32)

    xp, st = _tc_proj(x2, W, att2)
    st_flat = jnp.concatenate(
        [st.reshape(2 * N), jnp.zeros((STP - 2 * N,), F32)])
    sep, dp, bp = _k1(nidx2, eidx2, st_flat, ones2, zeros1)
    eexp2, denp = _k2(nidx2, eidx2, st_flat, ones2, zeros1, sep)
    alpha2 = _k2b(nidx2, eexp2, denp)
    nidx64 = nidx2.reshape(ER64, CH)
    eidx64 = eidx2.reshape(ER64, CH)
    alpha64 = alpha2.reshape(ER64, CH)
    zhalf = jnp.zeros((NPAD, FH), F32)
    xp_pad = jnp.concatenate([xp, jnp.zeros((NPAD - N, F), F32)])
    halves = []
    for h in range(2):
        tab = xp_pad[:, h * FH:(h + 1) * FH]
        xep = _k34(nidx64, eidx64, alpha64, tab, zhalf)
        xedge = _tc_comb(xep, bp)
        xnp_ = _k34(eidx64, nidx64, alpha64, xedge, zhalf)
        halves.append(_tc_comb(xnp_, dp))
    xnode = jnp.concatenate(halves, axis=1)
    return xnode[:N].reshape(1, N, F)


# speedup vs baseline: 1.1301x; 1.1289x over previous
import functools

import jax
import jax.numpy as jnp
from jax import lax
from jax.experimental import pallas as pl
from jax.experimental.pallas import tpu as pltpu
from jax.experimental.pallas import tpu_sc as plsc

N = 10000
M = 10000
E = 320000
F = 128
NEG = 0.2
NC, NS = 2, 16
NW = NC * NS
NR = 80
EPT = NR * 128
E_PAD = NW * EPT
ER = E_PAD // 128
NPAD = 10240
CHUNK = NPAD // NS
STP = 20480
F32 = jnp.float32
I32 = jnp.int32


def _mesh():
    return plsc.VectorSubcoreMesh(core_axis_name="c", subcore_axis_name="s")


def _wid_r0_off():
    cid = lax.axis_index("c")
    sid = lax.axis_index("s")
    wid = cid * NS + sid
    return cid, sid, wid, wid * NR, sid * CHUNK


def _tc_proj_body(x_ref, w_ref, att_ref, xp_ref, st_ref):
    xp = jnp.dot(x_ref[...], w_ref[...], preferred_element_type=F32)
    xp_ref[...] = xp
    st_ref[...] = lax.dot_general(
        xp, att_ref[...], (((1,), (1,)), ((), ())), preferred_element_type=F32
    )


def _tc_proj(x2, w, att2):
    return pl.pallas_call(
        _tc_proj_body,
        out_shape=(
            jax.ShapeDtypeStruct((N, F), F32),
            jax.ShapeDtypeStruct((N, 2), F32),
        ),
    )(x2, w, att2)


def _k1_body(nidx_h, eidx_h, st_h, ones_h, zeros_h,
             sep_h, dp_h, bp_h,
             idxn_v, idxe_v, st_v, ones_v, tv_v, se_sh, d_sh, b_sh):
    cid, sid, wid, r0, off = _wid_r0_off()
    zsl = pl.ds(off, CHUNK)
    pltpu.sync_copy(zeros_h.at[zsl], se_sh.at[zsl])
    pltpu.sync_copy(zeros_h.at[zsl], d_sh.at[zsl])
    pltpu.sync_copy(zeros_h.at[zsl], b_sh.at[zsl])
    pltpu.sync_copy(nidx_h.at[pl.ds(r0, NR), :], idxn_v)
    pltpu.sync_copy(eidx_h.at[pl.ds(r0, NR), :], idxe_v)
    pltpu.sync_copy(st_h, st_v)
    pltpu.sync_copy(ones_h.at[pl.ds(r0, NR), :], ones_v)

    def body(i, carry):
        for q in range(8):
            sl = pl.ds(q * 16, 16)
            iv = idxn_v[i, sl]
            tv = plsc.load_gather(st_v, [iv * 2 + 1])
            tv_v[i, sl] = tv * ones_v[i, sl]
        return carry

    lax.fori_loop(0, NR, body, 0)
    plsc.subcore_barrier()

    def scat(i, carry):
        pltpu.sync_copy(tv_v.at[i], se_sh.at[idxe_v.at[i]], add=True)
        pltpu.sync_copy(ones_v.at[i], d_sh.at[idxn_v.at[i]], add=True)
        pltpu.sync_copy(ones_v.at[i], b_sh.at[idxe_v.at[i]], add=True)
        return carry

    lax.fori_loop(0, NR, scat, 0)
    plsc.subcore_barrier()
    osl = pl.ds(cid * NPAD + off, CHUNK)
    pltpu.sync_copy(se_sh.at[zsl], sep_h.at[osl])
    pltpu.sync_copy(d_sh.at[zsl], dp_h.at[osl])
    pltpu.sync_copy(b_sh.at[zsl], bp_h.at[osl])


def _k1(nidx2, eidx2, st_flat, ones2, zeros1):
    return pl.kernel(
        _k1_body,
        out_type=(
            jax.ShapeDtypeStruct((NC * NPAD,), F32),
            jax.ShapeDtypeStruct((NC * NPAD,), F32),
            jax.ShapeDtypeStruct((NC * NPAD,), F32),
        ),
        mesh=_mesh(),
        compiler_params=pltpu.CompilerParams(needs_layout_passes=False),
        scratch_types=[
            pltpu.VMEM((NR, 128), I32),
            pltpu.VMEM((NR, 128), I32),
            pltpu.VMEM((STP,), F32),
            pltpu.VMEM((NR, 128), F32),
            pltpu.VMEM((NR, 128), F32),
            pltpu.VMEM_SHARED((NPAD,), F32),
            pltpu.VMEM_SHARED((NPAD,), F32),
            pltpu.VMEM_SHARED((NPAD,), F32),
        ],
    )(nidx2, eidx2, st_flat, ones2, zeros1)


def _k2_body(nidx_h, eidx_h, st_h, ones_h, zeros_h, sep_h,
             eexp_h, denp_h,
             idxn_v, idxe_v, st_v, ones_v, se0_v, se1_v, ex_v, den_sh):
    cid, sid, wid, r0, off = _wid_r0_off()
    zsl = pl.ds(off, CHUNK)
    pltpu.sync_copy(zeros_h.at[zsl], den_sh.at[zsl])
    pltpu.sync_copy(nidx_h.at[pl.ds(r0, NR), :], idxn_v)
    pltpu.sync_copy(eidx_h.at[pl.ds(r0, NR), :], idxe_v)
    pltpu.sync_copy(st_h, st_v)
    pltpu.sync_copy(ones_h.at[pl.ds(r0, NR), :], ones_v)
    pltpu.sync_copy(sep_h.at[pl.ds(0, NPAD)], se0_v)
    pltpu.sync_copy(sep_h.at[pl.ds(NPAD, NPAD)], se1_v)

    def comb(i, carry):
        sl = pl.ds(i * 16, 16)
        se0_v[sl] = se0_v[sl] + se1_v[sl]
        return carry

    lax.fori_loop(0, NPAD // 16, comb, 0)

    def body(i, carry):
        for q in range(8):
            sl = pl.ds(q * 16, 16)
            ivn = idxn_v[i, sl]
            ive = idxe_v[i, sl]
            s = plsc.load_gather(st_v, [ivn * 2])
            se = plsc.load_gather(se0_v, [ive])
            e = s + se
            e = jnp.where(e > 0, e, NEG * e)
            ex_v[i, sl] = jnp.exp(e) * ones_v[i, sl]
        return carry

    lax.fori_loop(0, NR, body, 0)
    plsc.subcore_barrier()

    def scat(i, carry):
        pltpu.sync_copy(ex_v.at[i], den_sh.at[idxn_v.at[i]], add=True)
        return carry

    lax.fori_loop(0, NR, scat, 0)
    plsc.subcore_barrier()
    pltpu.sync_copy(ex_v, eexp_h.at[pl.ds(r0, NR), :])
    pltpu.sync_copy(den_sh.at[zsl], denp_h.at[pl.ds(cid * NPAD + off, CHUNK)])


def _k2(nidx2, eidx2, st_flat, ones2, zeros1, sep):
    return pl.kernel(
        _k2_body,
        out_type=(
            jax.ShapeDtypeStruct((ER, 128), F32),
            jax.ShapeDtypeStruct((NC * NPAD,), F32),
        ),
        mesh=_mesh(),
        compiler_params=pltpu.CompilerParams(needs_layout_passes=False),
        scratch_types=[
            pltpu.VMEM((NR, 128), I32),
            pltpu.VMEM((NR, 128), I32),
            pltpu.VMEM((STP,), F32),
            pltpu.VMEM((NR, 128), F32),
            pltpu.VMEM((NPAD,), F32),
            pltpu.VMEM((NPAD,), F32),
            pltpu.VMEM((NR, 128), F32),
            pltpu.VMEM_SHARED((NPAD,), F32),
        ],
    )(nidx2, eidx2, st_flat, ones2, zeros1, sep)


def _k2b_body(nidx_h, eexp_h, denp_h,
              alpha_h,
              idxn_v, den0_v, den1_v, al_v):
    cid, sid, wid, r0, off = _wid_r0_off()
    pltpu.sync_copy(nidx_h.at[pl.ds(r0, NR), :], idxn_v)
    pltpu.sync_copy(denp_h.at[pl.ds(0, NPAD)], den0_v)
    pltpu.sync_copy(denp_h.at[pl.ds(NPAD, NPAD)], den1_v)

    def comb(i, carry):
        sl = pl.ds(i * 16, 16)
        d = den0_v[sl] + den1_v[sl]
        den0_v[sl] = 1.0 / jnp.maximum(d, 1e-16)
        return carry

    lax.fori_loop(0, NPAD // 16, comb, 0)
    pltpu.sync_copy(eexp_h.at[pl.ds(r0, NR), :], al_v)

    def abody(i, carry):
        for q in range(8):
            sl = pl.ds(q * 16, 16)
            ivn = idxn_v[i, sl]
            al_v[i, sl] = al_v[i, sl] * plsc.load_gather(den0_v, [ivn])
        return carry

    lax.fori_loop(0, NR, abody, 0)
    pltpu.sync_copy(al_v, alpha_h.at[pl.ds(r0, NR), :])


def _k2b(nidx2, eexp2, denp):
    return pl.kernel(
        _k2b_body,
        out_type=jax.ShapeDtypeStruct((ER, 128), F32),
        mesh=_mesh(),
        compiler_params=pltpu.CompilerParams(needs_layout_passes=False),
        scratch_types=[
            pltpu.VMEM((NR, 128), I32),
            pltpu.VMEM((NPAD,), F32),
            pltpu.VMEM((NPAD,), F32),
            pltpu.VMEM((NR, 128), F32),
        ],
    )(nidx2, eexp2, denp)


CH = 64
NB = 4
SGC = 16
NCH = EPT // CH
NSC = NCH // SGC
ER64 = E_PAD // CH
FH = F // 2
NSC0 = 14
NSC1 = 2 * NSC - NSC0


def _row_pass_body(gidx_h, sidx_h, alpha_h, table_h, zbig_h,
                   out_h,
                   gi_v, si_v, al_v,
                   rows0_v, rows1_v, rows2_v, rows3_v, acc_sh,
                   gs0, gs1, gs2, gs3, ss0, ss1, ss2, ss3):
    cid, sid, wid, r0, off = _wid_r0_off()
    del wid, r0
    nsc_mine = jnp.where(cid == 0, NSC0, NSC1)
    c0 = jnp.where(cid == 0, sid * (NSC0 * SGC),
                   NS * NSC0 * SGC + sid * (NSC1 * SGC))
    zsl = pl.ds(off, CHUNK)
    pltpu.sync_copy(zbig_h.at[zsl, :], acc_sh.at[zsl, :])
    plsc.subcore_barrier()
    rows = (rows0_v, rows1_v, rows2_v, rows3_v)
    gsems = (gs0, gs1, gs2, gs3)
    ssems = (ss0, ss1, ss2, ss3)

    def wait_scatter(b):
        pltpu.make_async_copy(rows[b], acc_sh.at[si_v.at[0]], ssems[b]).wait()

    def issue_gather(j):
        pltpu.async_copy(table_h.at[gi_v.at[j]], rows[j % NB], gsems[j % NB])

    def sbody(g, carry):
        @pl.when(g > 0)
        def _():
            for b in range(NB):
                wait_scatter(b)

        rg = c0 + g * SGC
        pltpu.sync_copy(gidx_h.at[pl.ds(rg, SGC), :], gi_v)
        pltpu.sync_copy(sidx_h.at[pl.ds(rg, SGC), :], si_v)
        pltpu.sync_copy(alpha_h.at[pl.ds(rg, SGC), :], al_v)
        for j in range(NB - 1):
            issue_gather(j)
        for j in range(SGC):
            b = j % NB
            if j + NB - 1 < SGC:
                if j >= 1:
                    wait_scatter((j - 1) % NB)
                issue_gather(j + NB - 1)
            pltpu.make_async_copy(
                table_h.at[gi_v.at[j]], rows[b], gsems[b]).wait()

            def scale(r, c2):
                w = plsc.load_gather(
                    al_v, [jnp.full((16,), j, I32), jnp.full((16,), r, I32)])
                for q in range(FH // 16):
                    sl = pl.ds(q * 16, 16)
                    rows[b][r, sl] = rows[b][r, sl] * w
                return c2

            lax.fori_loop(0, CH, scale, 0)
            pltpu.async_copy(rows[b], acc_sh.at[si_v.at[j]], ssems[b],
                             add=True)
        return carry

    lax.fori_loop(0, nsc_mine, sbody, 0)
    for b in range(NB):
        wait_scatter(b)
    plsc.subcore_barrier()
    pltpu.sync_copy(acc_sh.at[zsl, :], out_h.at[cid, zsl, :])


def _k34(gidx64, sidx64, alpha64, table, zbig):
    return pl.kernel(
        _row_pass_body,
        out_type=jax.ShapeDtypeStruct((NC, NPAD, FH), F32),
        mesh=_mesh(),
        compiler_params=pltpu.CompilerParams(
            needs_layout_passes=False, use_tc_tiling_on_sc=False),
        scratch_types=[
            pltpu.VMEM((SGC, CH), I32),
            pltpu.VMEM((SGC, CH), I32),
            pltpu.VMEM((SGC, CH), F32),
            pltpu.VMEM((CH, FH), F32),
            pltpu.VMEM((CH, FH), F32),
            pltpu.VMEM((CH, FH), F32),
            pltpu.VMEM((CH, FH), F32),
            pltpu.VMEM_SHARED((NPAD, FH), F32),
            pltpu.SemaphoreType.DMA,
            pltpu.SemaphoreType.DMA,
            pltpu.SemaphoreType.DMA,
            pltpu.SemaphoreType.DMA,
            pltpu.SemaphoreType.DMA,
            pltpu.SemaphoreType.DMA,
            pltpu.SemaphoreType.DMA,
            pltpu.SemaphoreType.DMA,
        ],
    )(gidx64, sidx64, alpha64, table, zbig)


def _tc_comb_body(part_ref, deg_ref, out_ref):
    deg = deg_ref[pl.ds(0, NPAD)] + deg_ref[pl.ds(NPAD, NPAD)]
    inv = jnp.where(deg > 0, 1.0 / deg, 0.0)
    out_ref[...] = (part_ref[0] + part_ref[1]) * inv[:, None]


def _tc_comb(part, degp):
    return pl.pallas_call(
        _tc_comb_body,
        out_shape=jax.ShapeDtypeStruct((NPAD, part.shape[-1]), F32),
    )(part, degp)


def kernel(x, hyperedge_index, W, att):
    x2 = x.reshape(N, F)
    att2 = att.reshape(2, F)
    pad = E_PAD - E
    nidx2 = jnp.concatenate(
        [hyperedge_index[0], jnp.zeros((pad,), I32)]).reshape(ER, 128)
    eidx2 = jnp.concatenate(
        [hyperedge_index[1], jnp.zeros((pad,), I32)]).reshape(ER, 128)
    ones2 = jnp.concatenate(
        [jnp.ones((E,), F32), jnp.zeros((pad,), F32)]).reshape(ER, 128)
    zeros1 = jnp.zeros((NPAD,), F32)
    zbig = jnp.zeros((NPAD, F), F32)

    xp, st = _tc_proj(x2, W, att2)
    st_flat = jnp.concatenate(
        [st.reshape(2 * N), jnp.zeros((STP - 2 * N,), F32)])
    sep, dp, bp = _k1(nidx2, eidx2, st_flat, ones2, zeros1)
    eexp2, denp = _k2(nidx2, eidx2, st_flat, ones2, zeros1, sep)
    alpha2 = _k2b(nidx2, eexp2, denp)
    nidx64 = nidx2.reshape(ER64, CH)
    eidx64 = eidx2.reshape(ER64, CH)
    alpha64 = alpha2.reshape(ER64, CH)
    zhalf = jnp.zeros((NPAD, FH), F32)
    xp_pad = jnp.concatenate([xp, jnp.zeros((NPAD - N, F), F32)])
    halves = []
    for h in range(2):
        tab = xp_pad[:, h * FH:(h + 1) * FH]
        xep = _k34(nidx64, eidx64, alpha64, tab, zhalf)
        xedge = _tc_comb(xep, bp)
        xnp_ = _k34(eidx64, nidx64, alpha64, xedge, zhalf)
        halves.append(_tc_comb(xnp_, dp))
    xnode = jnp.concatenate(halves, axis=1)
    return xnode[:N].reshape(1, N, F)


# speedup vs baseline: 1.2784x; 1.1312x over previous
import functools

import jax
import jax.numpy as jnp
from jax import lax
from jax.experimental import pallas as pl
from jax.experimental.pallas import tpu as pltpu
from jax.experimental.pallas import tpu_sc as plsc

N = 10000
M = 10000
E = 320000
F = 128
NEG = 0.2
NC, NS = 2, 16
NW = NC * NS
NR = 80
EPT = NR * 128
E_PAD = NW * EPT
ER = E_PAD // 128
NPAD = 10240
CHUNK = NPAD // NS
STP = 20480
F32 = jnp.float32
I32 = jnp.int32


def _mesh():
    return plsc.VectorSubcoreMesh(core_axis_name="c", subcore_axis_name="s")


def _wid_r0_off():
    cid = lax.axis_index("c")
    sid = lax.axis_index("s")
    wid = cid * NS + sid
    return cid, sid, wid, wid * NR, sid * CHUNK


def _tc_proj_body(x_ref, w_ref, att_ref, xp_ref, st_ref):
    xp = jnp.dot(x_ref[...], w_ref[...], preferred_element_type=F32)
    xp_ref[...] = xp
    st_ref[...] = lax.dot_general(
        xp, att_ref[...], (((1,), (1,)), ((), ())), preferred_element_type=F32
    )


def _tc_proj(x2, w, att2):
    return pl.pallas_call(
        _tc_proj_body,
        out_shape=(
            jax.ShapeDtypeStruct((N, F), F32),
            jax.ShapeDtypeStruct((N, 2), F32),
        ),
    )(x2, w, att2)


def _k1_body(nidx_h, eidx_h, st_h, ones_h, zeros_h,
             sep_h, dp_h, bp_h,
             idxn_v, idxe_v, st_v, ones_v, tv_v, se_sh, d_sh, b_sh):
    cid, sid, wid, r0, off = _wid_r0_off()
    zsl = pl.ds(off, CHUNK)
    pltpu.sync_copy(zeros_h.at[zsl], se_sh.at[zsl])
    pltpu.sync_copy(zeros_h.at[zsl], d_sh.at[zsl])
    pltpu.sync_copy(zeros_h.at[zsl], b_sh.at[zsl])
    pltpu.sync_copy(nidx_h.at[pl.ds(r0, NR), :], idxn_v)
    pltpu.sync_copy(eidx_h.at[pl.ds(r0, NR), :], idxe_v)
    pltpu.sync_copy(st_h, st_v)
    pltpu.sync_copy(ones_h.at[pl.ds(r0, NR), :], ones_v)

    def body(i, carry):
        for q in range(8):
            sl = pl.ds(q * 16, 16)
            iv = idxn_v[i, sl]
            tv = plsc.load_gather(st_v, [iv * 2 + 1])
            tv_v[i, sl] = tv * ones_v[i, sl]
        return carry

    lax.fori_loop(0, NR, body, 0)
    plsc.subcore_barrier()

    def scat(i, carry):
        pltpu.sync_copy(tv_v.at[i], se_sh.at[idxe_v.at[i]], add=True)
        pltpu.sync_copy(ones_v.at[i], d_sh.at[idxn_v.at[i]], add=True)
        pltpu.sync_copy(ones_v.at[i], b_sh.at[idxe_v.at[i]], add=True)
        return carry

    lax.fori_loop(0, NR, scat, 0)
    plsc.subcore_barrier()
    osl = pl.ds(cid * NPAD + off, CHUNK)
    pltpu.sync_copy(se_sh.at[zsl], sep_h.at[osl])
    pltpu.sync_copy(d_sh.at[zsl], dp_h.at[osl])
    pltpu.sync_copy(b_sh.at[zsl], bp_h.at[osl])


def _k1(nidx2, eidx2, st_flat, ones2, zeros1):
    return pl.kernel(
        _k1_body,
        out_type=(
            jax.ShapeDtypeStruct((NC * NPAD,), F32),
            jax.ShapeDtypeStruct((NC * NPAD,), F32),
            jax.ShapeDtypeStruct((NC * NPAD,), F32),
        ),
        mesh=_mesh(),
        compiler_params=pltpu.CompilerParams(needs_layout_passes=False),
        scratch_types=[
            pltpu.VMEM((NR, 128), I32),
            pltpu.VMEM((NR, 128), I32),
            pltpu.VMEM((STP,), F32),
            pltpu.VMEM((NR, 128), F32),
            pltpu.VMEM((NR, 128), F32),
            pltpu.VMEM_SHARED((NPAD,), F32),
            pltpu.VMEM_SHARED((NPAD,), F32),
            pltpu.VMEM_SHARED((NPAD,), F32),
        ],
    )(nidx2, eidx2, st_flat, ones2, zeros1)


def _k2_body(nidx_h, eidx_h, st_h, ones_h, zeros_h, sep_h,
             eexp_h, denp_h,
             idxn_v, idxe_v, st_v, ones_v, se0_v, se1_v, ex_v, den_sh):
    cid, sid, wid, r0, off = _wid_r0_off()
    zsl = pl.ds(off, CHUNK)
    pltpu.sync_copy(zeros_h.at[zsl], den_sh.at[zsl])
    pltpu.sync_copy(nidx_h.at[pl.ds(r0, NR), :], idxn_v)
    pltpu.sync_copy(eidx_h.at[pl.ds(r0, NR), :], idxe_v)
    pltpu.sync_copy(st_h, st_v)
    pltpu.sync_copy(ones_h.at[pl.ds(r0, NR), :], ones_v)
    pltpu.sync_copy(sep_h.at[pl.ds(0, NPAD)], se0_v)
    pltpu.sync_copy(sep_h.at[pl.ds(NPAD, NPAD)], se1_v)

    def comb(i, carry):
        sl = pl.ds(i * 16, 16)
        se0_v[sl] = se0_v[sl] + se1_v[sl]
        return carry

    lax.fori_loop(0, NPAD // 16, comb, 0)

    def body(i, carry):
        for q in range(8):
            sl = pl.ds(q * 16, 16)
            ivn = idxn_v[i, sl]
            ive = idxe_v[i, sl]
            s = plsc.load_gather(st_v, [ivn * 2])
            se = plsc.load_gather(se0_v, [ive])
            e = s + se
            e = jnp.where(e > 0, e, NEG * e)
            ex_v[i, sl] = jnp.exp(e) * ones_v[i, sl]
        return carry

    lax.fori_loop(0, NR, body, 0)
    plsc.subcore_barrier()

    def scat(i, carry):
        pltpu.sync_copy(ex_v.at[i], den_sh.at[idxn_v.at[i]], add=True)
        return carry

    lax.fori_loop(0, NR, scat, 0)
    plsc.subcore_barrier()
    pltpu.sync_copy(ex_v, eexp_h.at[pl.ds(r0, NR), :])
    pltpu.sync_copy(den_sh.at[zsl], denp_h.at[pl.ds(cid * NPAD + off, CHUNK)])


def _k2(nidx2, eidx2, st_flat, ones2, zeros1, sep):
    return pl.kernel(
        _k2_body,
        out_type=(
            jax.ShapeDtypeStruct((ER, 128), F32),
            jax.ShapeDtypeStruct((NC * NPAD,), F32),
        ),
        mesh=_mesh(),
        compiler_params=pltpu.CompilerParams(needs_layout_passes=False),
        scratch_types=[
            pltpu.VMEM((NR, 128), I32),
            pltpu.VMEM((NR, 128), I32),
            pltpu.VMEM((STP,), F32),
            pltpu.VMEM((NR, 128), F32),
            pltpu.VMEM((NPAD,), F32),
            pltpu.VMEM((NPAD,), F32),
            pltpu.VMEM((NR, 128), F32),
            pltpu.VMEM_SHARED((NPAD,), F32),
        ],
    )(nidx2, eidx2, st_flat, ones2, zeros1, sep)


def _k2b_body(nidx_h, eexp_h, denp_h,
              alpha_h,
              idxn_v, den0_v, den1_v, al_v):
    cid, sid, wid, r0, off = _wid_r0_off()
    pltpu.sync_copy(nidx_h.at[pl.ds(r0, NR), :], idxn_v)
    pltpu.sync_copy(denp_h.at[pl.ds(0, NPAD)], den0_v)
    pltpu.sync_copy(denp_h.at[pl.ds(NPAD, NPAD)], den1_v)

    def comb(i, carry):
        sl = pl.ds(i * 16, 16)
        d = den0_v[sl] + den1_v[sl]
        den0_v[sl] = 1.0 / jnp.maximum(d, 1e-16)
        return carry

    lax.fori_loop(0, NPAD // 16, comb, 0)
    pltpu.sync_copy(eexp_h.at[pl.ds(r0, NR), :], al_v)

    def abody(i, carry):
        for q in range(8):
            sl = pl.ds(q * 16, 16)
            ivn = idxn_v[i, sl]
            al_v[i, sl] = al_v[i, sl] * plsc.load_gather(den0_v, [ivn])
        return carry

    lax.fori_loop(0, NR, abody, 0)
    pltpu.sync_copy(al_v, alpha_h.at[pl.ds(r0, NR), :])


def _k2b(nidx2, eexp2, denp):
    return pl.kernel(
        _k2b_body,
        out_type=jax.ShapeDtypeStruct((ER, 128), F32),
        mesh=_mesh(),
        compiler_params=pltpu.CompilerParams(needs_layout_passes=False),
        scratch_types=[
            pltpu.VMEM((NR, 128), I32),
            pltpu.VMEM((NPAD,), F32),
            pltpu.VMEM((NPAD,), F32),
            pltpu.VMEM((NR, 128), F32),
        ],
    )(nidx2, eexp2, denp)


CH = 64
NB = 4
SGC = 16
NCH = EPT // CH
NSC = NCH // SGC
ER64 = E_PAD // CH
FH = F // 2
NSC0 = 14
NSC1 = 2 * NSC - NSC0


def _row_pass_body(gidx_h, sidx_h, alpha_h, table_h, zbig_h,
                   out_h,
                   gi_v, si_v, al_v,
                   rows0_v, rows1_v, rows2_v, rows3_v, acc_sh,
                   gs0, gs1, gs2, gs3, ss0, ss1, ss2, ss3):
    cid, sid, wid, r0, off = _wid_r0_off()
    del wid, r0
    nsc_mine = jnp.where(cid == 0, NSC0, NSC1)
    c0 = jnp.where(cid == 0, sid * (NSC0 * SGC),
                   NS * NSC0 * SGC + sid * (NSC1 * SGC))
    zsl = pl.ds(off, CHUNK)
    pltpu.sync_copy(zbig_h.at[zsl, :], acc_sh.at[zsl, :])
    plsc.subcore_barrier()
    rows = (rows0_v, rows1_v, rows2_v, rows3_v)
    gsems = (gs0, gs1, gs2, gs3)
    ssems = (ss0, ss1, ss2, ss3)

    def wait_scatter(b):
        pltpu.make_async_copy(rows[b], acc_sh.at[si_v.at[0]], ssems[b]).wait()

    def issue_gather(j):
        pltpu.async_copy(table_h.at[gi_v.at[j]], rows[j % NB], gsems[j % NB])

    def sbody(g, carry):
        @pl.when(g > 0)
        def _():
            for b in range(NB):
                wait_scatter(b)

        rg = c0 + g * SGC
        pltpu.sync_copy(gidx_h.at[pl.ds(rg, SGC), :], gi_v)
        pltpu.sync_copy(sidx_h.at[pl.ds(rg, SGC), :], si_v)
        pltpu.sync_copy(alpha_h.at[pl.ds(rg, SGC), :], al_v)
        for j in range(NB - 1):
            issue_gather(j)
        for j in range(SGC):
            b = j % NB
            if j + NB - 1 < SGC:
                if j >= 1:
                    wait_scatter((j - 1) % NB)
                issue_gather(j + NB - 1)
            pltpu.make_async_copy(
                table_h.at[gi_v.at[j]], rows[b], gsems[b]).wait()

            def scale(r, c2):
                w = plsc.load_gather(
                    al_v, [jnp.full((16,), j, I32), jnp.full((16,), r, I32)])
                for q in range(F // 16):
                    sl = pl.ds(q * 16, 16)
                    rows[b][r, sl] = rows[b][r, sl] * w
                return c2

            lax.fori_loop(0, CH, scale, 0)
            pltpu.async_copy(rows[b], acc_sh.at[si_v.at[j]], ssems[b],
                             add=True)
        return carry

    lax.fori_loop(0, nsc_mine, sbody, 0)
    for b in range(NB):
        wait_scatter(b)
    plsc.subcore_barrier()
    pltpu.sync_copy(acc_sh.at[zsl, :], out_h.at[cid, zsl, :])


def _k34(gidx64, sidx64, alpha64, table, zbig):
    return pl.kernel(
        _row_pass_body,
        out_type=jax.ShapeDtypeStruct((NC, NPAD, F), F32),
        mesh=_mesh(),
        compiler_params=pltpu.CompilerParams(needs_layout_passes=False),
        scratch_types=[
            pltpu.VMEM((SGC, CH), I32),
            pltpu.VMEM((SGC, CH), I32),
            pltpu.VMEM((SGC, CH), F32),
            pltpu.VMEM((CH, F), F32),
            pltpu.VMEM((CH, F), F32),
            pltpu.VMEM((CH, F), F32),
            pltpu.VMEM((CH, F), F32),
            pltpu.VMEM_SHARED((NPAD, F), F32),
            pltpu.SemaphoreType.DMA,
            pltpu.SemaphoreType.DMA,
            pltpu.SemaphoreType.DMA,
            pltpu.SemaphoreType.DMA,
            pltpu.SemaphoreType.DMA,
            pltpu.SemaphoreType.DMA,
            pltpu.SemaphoreType.DMA,
            pltpu.SemaphoreType.DMA,
        ],
    )(gidx64, sidx64, alpha64, table, zbig)


def _tc_comb_body(part_ref, deg_ref, out_ref):
    deg = deg_ref[pl.ds(0, NPAD)] + deg_ref[pl.ds(NPAD, NPAD)]
    inv = jnp.where(deg > 0, 1.0 / deg, 0.0)
    out_ref[...] = (part_ref[0] + part_ref[1]) * inv[:, None]


def _tc_comb(part, degp):
    return pl.pallas_call(
        _tc_comb_body,
        out_shape=jax.ShapeDtypeStruct((NPAD, part.shape[-1]), F32),
    )(part, degp)


def kernel(x, hyperedge_index, W, att):
    x2 = x.reshape(N, F)
    att2 = att.reshape(2, F)
    pad = E_PAD - E
    nidx2 = jnp.concatenate(
        [hyperedge_index[0], jnp.zeros((pad,), I32)]).reshape(ER, 128)
    eidx2 = jnp.concatenate(
        [hyperedge_index[1], jnp.zeros((pad,), I32)]).reshape(ER, 128)
    ones2 = jnp.concatenate(
        [jnp.ones((E,), F32), jnp.zeros((pad,), F32)]).reshape(ER, 128)
    zeros1 = jnp.zeros((NPAD,), F32)
    zbig = jnp.zeros((NPAD, F), F32)

    xp, st = _tc_proj(x2, W, att2)
    st_flat = jnp.concatenate(
        [st.reshape(2 * N), jnp.zeros((STP - 2 * N,), F32)])
    sep, dp, bp = _k1(nidx2, eidx2, st_flat, ones2, zeros1)
    eexp2, denp = _k2(nidx2, eidx2, st_flat, ones2, zeros1, sep)
    alpha2 = _k2b(nidx2, eexp2, denp)
    nidx64 = nidx2.reshape(ER64, CH)
    eidx64 = eidx2.reshape(ER64, CH)
    alpha64 = alpha2.reshape(ER64, CH)
    xp_pad = jnp.concatenate([xp, jnp.zeros((NPAD - N, F), F32)])
    xep = _k34(nidx64, eidx64, alpha64, xp_pad, zbig)
    xedge = _tc_comb(xep, bp)
    xnp_ = _k34(eidx64, nidx64, alpha64, xedge, zbig)
    xnode = _tc_comb(xnp_, dp)
    return xnode[:N].reshape(1, N, F)
